# Initial kernel scaffold; baseline (speedup 1.0000x reference)
import jax
import jax.numpy as jnp
from jax.experimental import pallas as pl


def kernel(x, seq_len, h0, w_ih0, w_hh0, b_ih0, b_hh0, w_ih1, w_hh1, b_ih1, b_hh1, ode_w0, ode_b0, ode_w1, ode_b1, fc_w, fc_b):
    raise NotImplementedError("write your pallas kernel here")


# speedup vs baseline: 1.1325x; 1.1325x over previous
import functools

import jax
import jax.numpy as jnp
from jax import lax
from jax.experimental import pallas as pl
from jax.experimental.pallas import tpu as pltpu

_DT = 0.1
_DISC = 10
_HSTEP = _DT / _DISC
_NSTEPS = _DISC - 1


def _fused_kernel(x_ref, h0_ref, wih0_ref, whh0_ref, b0_ref, wih1_ref,
                  whh1_ref, b1_ref, w0_ref, ob0_ref, w1_ref, ob1_ref,
                  fcw_ref, fcb_ref, out_ref, h0c, h1c, pre_buf, rnn_buf,
                  *, tc, bblk, rt, z):
    c = pl.program_id(1)
    rows = tc * bblk

    @pl.when(c == 0)
    def _init():
        h0c[:, :] = jnp.broadcast_to(h0_ref[0:1, :], (bblk, z))
        h1c[:, :] = jnp.broadcast_to(h0_ref[1:2, :], (bblk, z))

    h1_in = h1c[:, :]
    rnn_buf[(tc - 1) * bblk:tc * bblk, :] = h1_in

    x2d = x_ref[:, :, :].reshape(rows, x_ref.shape[2])
    pre_buf[:, :] = (jnp.dot(x2d, wih0_ref[:, :],
                             preferred_element_type=jnp.float32)
                     + b0_ref[:, :])

    whh0 = whh0_ref[:, :]
    wih1 = wih1_ref[:, :]
    whh1 = whh1_ref[:, :]
    b1 = b1_ref[:, :]

    def _cell(t, h0, h1):
        off = pl.multiple_of(t * bblk, bblk)
        p = pre_buf[pl.ds(off, bblk), :]
        h0n = jnp.tanh(p + jnp.dot(h0, whh0,
                                   preferred_element_type=jnp.float32))
        h1n = jnp.tanh(jnp.dot(h0n, wih1, preferred_element_type=jnp.float32)
                       + jnp.dot(h1, whh1,
                                 preferred_element_type=jnp.float32) + b1)
        return h0n, h1n

    def _rnn_body(i, carry):
        t = tc - 1 - i
        h0n, h1n = _cell(t, *carry)
        rnn_buf[pl.ds(pl.multiple_of((t - 1) * bblk, bblk), bblk), :] = h1n
        return (h0n, h1n)

    h0v, h1v = lax.fori_loop(0, tc - 1, _rnn_body, (h0c[:, :], h1_in))
    h0v, h1v = _cell(0, h0v, h1v)
    h0c[:, :] = h0v
    h1c[:, :] = h1v

    w0 = w0_ref[:, :]
    ob0 = ob0_ref[:, :]
    w1 = w1_ref[:, :]
    ob1 = ob1_ref[:, :]
    fcw = fcw_ref[:, :]
    fcb = fcb_ref[:, :]

    def _f(y):
        a = jax.nn.softplus(jnp.dot(y, w0,
                                    preferred_element_type=jnp.float32) + ob0)
        a = jax.nn.softplus(jnp.dot(a, w1,
                                    preferred_element_type=jnp.float32) + ob1)
        return jnp.dot(a, fcw, preferred_element_type=jnp.float32) + fcb

    tpr = rt // bblk

    def _tile_body(ri, _):
        off = pl.multiple_of(ri * rt, rt)
        y0 = rnn_buf[pl.ds(off, rt), :]

        def _step(_s, y):
            k1 = _f(y)
            k2 = _f(y + (0.5 * _HSTEP) * k1)
            k3 = _f(y + (0.5 * _HSTEP) * k2)
            k4 = _f(y + _HSTEP * k3)
            return y + (_HSTEP / 6.0) * (k1 + 2.0 * k2 + 2.0 * k3 + k4)

        y = lax.fori_loop(0, _NSTEPS, _step, y0)
        out_ref[pl.ds(ri * tpr, tpr), :, :] = y.reshape(tpr, bblk, z)
        return 0

    lax.fori_loop(0, rows // rt, _tile_body, 0)

    @pl.when(c == 0)
    def _passthrough():
        out_ref[tc - 1, :, :] = rnn_buf[(tc - 2) * bblk:(tc - 1) * bblk, :]


def kernel(x, seq_len, h0, w_ih0, w_hh0, b_ih0, b_hh0, w_ih1, w_hh1, b_ih1,
           b_hh1, ode_w0, ode_b0, ode_w1, ode_b1, fc_w, fc_b):
    B, T, inp = x.shape
    z = w_ih0.shape[0]
    h = ode_w0.shape[0]
    bblk = B // 2 if B % 2 == 0 else B
    tc = 64 if (T % 64 == 0 and T > 64) else T
    nc = T // tc
    rt = bblk * min(4, tc)

    xT = jnp.transpose(x, (1, 0, 2))
    args = (
        xT,
        h0.reshape(2, z),
        w_ih0.T, w_hh0.T, (b_ih0 + b_hh0).reshape(1, z),
        w_ih1.T, w_hh1.T, (b_ih1 + b_hh1).reshape(1, z),
        ode_w0.T, ode_b0.reshape(1, h),
        ode_w1.T, ode_b1.reshape(1, h),
        fc_w.T, fc_b.reshape(1, z),
    )
    full = lambda a: pl.BlockSpec(a.shape, lambda i, c: (0,) * a.ndim)
    grid = (B // bblk, nc)
    out_t = pl.pallas_call(
        functools.partial(_fused_kernel, tc=tc, bblk=bblk, rt=rt, z=z),
        grid=grid,
        in_specs=[pl.BlockSpec((tc, bblk, inp),
                               lambda i, c: (nc - 1 - c, i, 0))]
                 + [full(a) for a in args[1:]],
        out_specs=pl.BlockSpec((tc, bblk, z), lambda i, c: (nc - 1 - c, i, 0)),
        out_shape=jax.ShapeDtypeStruct((T, B, z), jnp.float32),
        scratch_shapes=[
            pltpu.VMEM((bblk, z), jnp.float32),
            pltpu.VMEM((bblk, z), jnp.float32),
            pltpu.VMEM((tc * bblk, z), jnp.float32),
            pltpu.VMEM((tc * bblk, z), jnp.float32),
        ],
        compiler_params=pltpu.CompilerParams(
            dimension_semantics=("parallel", "arbitrary"),
            vmem_limit_bytes=56 * 1024 * 1024,
        ),
        name="ode_encoder_fused",
    )(*args)
    return jnp.transpose(out_t, (1, 0, 2))
